# Initial kernel scaffold; baseline (speedup 1.0000x reference)
import jax
import jax.numpy as jnp
from jax.experimental import pallas as pl


def kernel(target_ids, input_ids, word_embeddings, position_embeddings, gamma, beta):
    raise NotImplementedError("write your pallas kernel here")


# speedup vs baseline: 1.0189x; 1.0189x over previous
import functools
import jax
import jax.numpy as jnp
from jax import lax
from jax.experimental import pallas as pl
from jax.experimental.pallas import tpu as pltpu
from jax.experimental.pallas import tpu_sc as plsc

H = 128
LANES = 16
HV = H // LANES
NC = 2
NS = 16
NW = NC * NS


def _sc_gather_pool(tgt_idx, msa_idx, table, B, L, n_msa):
    total = B * L
    P = total // NW
    C = 128
    n_chunks = P // C
    inv_n = 1.0 / n_msa

    mesh = plsc.VectorSubcoreMesh(core_axis_name="c", subcore_axis_name="s")

    @functools.partial(
        pl.kernel,
        out_type=(
            jax.ShapeDtypeStruct((total, H), jnp.float32),
            jax.ShapeDtypeStruct((total, H), jnp.float32),
        ),
        mesh=mesh,
        scratch_types=[
            pltpu.VMEM((C,), jnp.int32),
            pltpu.VMEM((C,), jnp.int32),
            pltpu.VMEM((C, H), jnp.float32),
            pltpu.VMEM((C, H), jnp.float32),
            pltpu.VMEM((C, H), jnp.float32),
            pltpu.SemaphoreType.DMA,
        ],
    )
    def k(tgt_idx_hbm, msa_idx_hbm, table_hbm, s_out, m_out,
          tidx_v, midx_v, trows, mbuf, acc, sem):
        wid = lax.axis_index("s") * NC + lax.axis_index("c")
        base = wid * P
        b = base // L
        l0 = base - b * L
        for ci in range(n_chunks):
            pos0 = base + ci * C
            pltpu.sync_copy(tgt_idx_hbm.at[pl.ds(pos0, C)], tidx_v)
            pltpu.async_copy(table_hbm.at[tidx_v], trows, sem).wait()
            src0 = b * (n_msa * L) + l0 + ci * C
            pltpu.sync_copy(msa_idx_hbm.at[pl.ds(src0, C)], midx_v)
            pltpu.async_copy(table_hbm.at[midx_v], acc, sem).wait()
            for j in range(1, n_msa):
                pltpu.sync_copy(msa_idx_hbm.at[pl.ds(src0 + j * L, C)], midx_v)
                pltpu.async_copy(table_hbm.at[midx_v], mbuf, sem).wait()

                def add_row(p, _):
                    for h in range(HV):
                        plsc.addupdate(acc.at[p, pl.ds(h * LANES, LANES)],
                                       mbuf[p, pl.ds(h * LANES, LANES)])
                    return 0

                lax.fori_loop(0, C, add_row, 0)

            def fin_row(p, _):
                for h in range(HV):
                    m = acc[p, pl.ds(h * LANES, LANES)] * inv_n
                    acc[p, pl.ds(h * LANES, LANES)] = m
                    plsc.addupdate(trows.at[p, pl.ds(h * LANES, LANES)], m)
                return 0

            lax.fori_loop(0, C, fin_row, 0)
            pltpu.sync_copy(acc, m_out.at[pl.ds(pos0, C)])
            pltpu.sync_copy(trows, s_out.at[pl.ds(pos0, C)])

    return k(tgt_idx, msa_idx, table)


def _ln_body(s_ref, pos_ref, gamma_ref, beta_ref, out_ref):
    x = s_ref[0] + pos_ref[...]
    mean = jnp.mean(x, axis=-1, keepdims=True)
    cx = x - mean
    var = jnp.mean(cx * cx, axis=-1, keepdims=True)
    inv = lax.rsqrt(var + 1e-12)
    out_ref[0] = cx * inv * gamma_ref[0] + beta_ref[0]


def kernel(target_ids, input_ids, word_embeddings, position_embeddings, gamma, beta):
    B, L = target_ids.shape
    n_msa = input_ids.shape[1]
    tgt_idx = target_ids.reshape(-1).astype(jnp.int32)
    msa_idx = input_ids.reshape(-1).astype(jnp.int32)
    s, msa_mean = _sc_gather_pool(tgt_idx, msa_idx, word_embeddings, B, L, n_msa)
    pos = position_embeddings[:L]
    emb = pl.pallas_call(
        _ln_body,
        grid=(B,),
        in_specs=[
            pl.BlockSpec((1, L, H), lambda b: (b, 0, 0)),
            pl.BlockSpec((L, H), lambda b: (0, 0)),
            pl.BlockSpec((1, H), lambda b: (0, 0)),
            pl.BlockSpec((1, H), lambda b: (0, 0)),
        ],
        out_specs=pl.BlockSpec((1, L, H), lambda b: (b, 0, 0)),
        out_shape=jax.ShapeDtypeStruct((B, L, H), jnp.float32),
    )(s.reshape(B, L, H), pos, gamma.reshape(1, H), beta.reshape(1, H))
    return emb, msa_mean.reshape(B, L, H)


# speedup vs baseline: 1.4892x; 1.4616x over previous
import functools
import jax
import jax.numpy as jnp
from jax import lax
from jax.experimental import pallas as pl
from jax.experimental.pallas import tpu as pltpu
from jax.experimental.pallas import tpu_sc as plsc

H = 128
LANES = 16
HV = H // LANES
NC = 2
NS = 16
NW = NC * NS
C = 128


def _sc_gather_pool(tgt_idx2d, msa_idx2d, table, B, L, n_msa):
    total = B * L
    P = total // NW
    n_chunks = P // C
    n_steps = n_chunks * n_msa
    inv_n = 1.0 / n_msa

    mesh = plsc.VectorSubcoreMesh(core_axis_name="c", subcore_axis_name="s")

    @functools.partial(
        pl.kernel,
        out_type=(
            jax.ShapeDtypeStruct((total, H), jnp.float32),
            jax.ShapeDtypeStruct((total, H), jnp.float32),
        ),
        mesh=mesh,
        scratch_types=[
            pltpu.VMEM((P,), jnp.int32),
            pltpu.VMEM((n_msa * P,), jnp.int32),
            pltpu.VMEM((P, H), jnp.float32),
            pltpu.VMEM((C, H), jnp.float32),
            pltpu.VMEM((C, H), jnp.float32),
            pltpu.VMEM((P, H), jnp.float32),
            pltpu.SemaphoreType.DMA,
            pltpu.SemaphoreType.DMA,
            pltpu.SemaphoreType.DMA,
            pltpu.SemaphoreType.DMA,
            pltpu.SemaphoreType.DMA,
        ],
    )
    def k(tgt_idx_hbm, msa_idx_hbm, table_hbm, s_out, m_out,
          tidx, midx, trows, buf0, buf1, acc,
          sem_i, sem_t, sem_a, sem_b0, sem_b1):
        wid = lax.axis_index("s") * NC + lax.axis_index("c")
        base = wid * P
        b = base // L
        l0 = base - b * L
        src0 = b * (n_msa * L) + l0

        icps = [pltpu.make_async_copy(
            tgt_idx_hbm.at[pl.ds(base, P)], tidx, sem_i)]
        for j in range(n_msa):
            icps.append(pltpu.make_async_copy(
                msa_idx_hbm.at[pl.ds(src0 + j * L, P)],
                midx.at[pl.ds(j * P, P)], sem_i))
        for cp in icps:
            cp.start()
        for cp in icps:
            cp.wait()

        tcps = []
        for ci in range(n_chunks):
            cp = pltpu.make_async_copy(
                table_hbm.at[tidx.at[pl.ds(ci * C, C)]],
                trows.at[pl.ds(ci * C, C)], sem_t)
            cp.start()
            tcps.append(cp)

        bufs = (buf0, buf1)
        bsems = (sem_b0, sem_b1)

        def fire(step):
            ci, j = divmod(step, n_msa)
            isl = midx.at[pl.ds(j * P + ci * C, C)]
            if j == 0:
                cp = pltpu.make_async_copy(
                    table_hbm.at[isl], acc.at[pl.ds(ci * C, C)], sem_a)
            else:
                cp = pltpu.make_async_copy(
                    table_hbm.at[isl], bufs[step % 2], bsems[step % 2])
            cp.start()
            return cp

        cps = {0: fire(0), 1: fire(1)}
        for step in range(n_steps):
            cps.pop(step).wait()
            ci, j = divmod(step, n_msa)
            if j > 0:
                src = bufs[step % 2]
                a0 = ci * C

                def add_row(p, _):
                    for h in range(HV):
                        plsc.addupdate(acc.at[a0 + p, pl.ds(h * LANES, LANES)],
                                       src[p, pl.ds(h * LANES, LANES)])
                    return 0

                lax.fori_loop(0, C, add_row, 0)
            if step + 2 < n_steps:
                cps[step + 2] = fire(step + 2)

        for cp in tcps:
            cp.wait()

        def fin_row(p, _):
            for h in range(HV):
                m = acc[p, pl.ds(h * LANES, LANES)] * inv_n
                acc[p, pl.ds(h * LANES, LANES)] = m
                plsc.addupdate(trows.at[p, pl.ds(h * LANES, LANES)], m)
            return 0

        lax.fori_loop(0, P, fin_row, 0)
        pltpu.sync_copy(acc, m_out.at[pl.ds(base, P)])
        pltpu.sync_copy(trows, s_out.at[pl.ds(base, P)])

    return k(tgt_idx2d, msa_idx2d, table)


def _ln_body(s_ref, pos_ref, gamma_ref, beta_ref, out_ref):
    x = s_ref[0] + pos_ref[...]
    mean = jnp.mean(x, axis=-1, keepdims=True)
    cx = x - mean
    var = jnp.mean(cx * cx, axis=-1, keepdims=True)
    inv = lax.rsqrt(var + 1e-12)
    out_ref[0] = cx * inv * gamma_ref[0] + beta_ref[0]


def kernel(target_ids, input_ids, word_embeddings, position_embeddings, gamma, beta):
    B, L = target_ids.shape
    n_msa = input_ids.shape[1]
    tgt_idx = target_ids.astype(jnp.int32).reshape(-1)
    msa_idx = input_ids.astype(jnp.int32).reshape(-1)
    s, msa_mean = _sc_gather_pool(tgt_idx, msa_idx, word_embeddings, B, L, n_msa)
    pos = position_embeddings[:L]
    emb = pl.pallas_call(
        _ln_body,
        grid=(B,),
        in_specs=[
            pl.BlockSpec((1, L, H), lambda b: (b, 0, 0)),
            pl.BlockSpec((L, H), lambda b: (0, 0)),
            pl.BlockSpec((1, H), lambda b: (0, 0)),
            pl.BlockSpec((1, H), lambda b: (0, 0)),
        ],
        out_specs=pl.BlockSpec((1, L, H), lambda b: (b, 0, 0)),
        out_shape=jax.ShapeDtypeStruct((B, L, H), jnp.float32),
    )(s.reshape(B, L, H), pos, gamma.reshape(1, H), beta.reshape(1, H))
    return emb, msa_mean.reshape(B, L, H)
